# Initial kernel scaffold; baseline (speedup 1.0000x reference)
import jax
import jax.numpy as jnp
from jax.experimental import pallas as pl


def kernel(x, emb, ln1_g, ln1_b, ln2_g, ln2_b, q_proj, k_proj, v_proj, out_proj, gate_W, gate_b, W1, b1, W2, b2, Wd, bd):
    raise NotImplementedError("write your pallas kernel here")


# speedup vs baseline: 1.0947x; 1.0947x over previous
import functools
import math

import jax
import jax.numpy as jnp
from jax import lax
from jax.experimental import pallas as pl
from jax.experimental.pallas import tpu as pltpu
from jax.experimental.pallas import tpu_sc as plsc

B, S, D = 1, 2048, 768
NH, DH = 12, 64
E, K, HID = 8, 2, 2048
MAX_LEN = 2048
HALF = DH // 2

NEG = -1e30


def _make_sc_gather(V, Dm, nrows, dtype):
  info = plsc.get_sparse_core_info()
  NC, NS = info.num_cores, info.num_subcores
  NW = NC * NS
  assert nrows % NW == 0
  b_per_w = nrows // NW
  mesh = plsc.VectorSubcoreMesh(core_axis_name="c", subcore_axis_name="s")

  @functools.partial(
      pl.kernel, mesh=mesh,
      out_type=jax.ShapeDtypeStruct((nrows, Dm), dtype),
      scratch_types=[
          pltpu.VMEM((b_per_w,), jnp.int32),
          pltpu.VMEM((b_per_w, Dm), dtype),
          pltpu.SemaphoreType.DMA,
      ],
  )
  def gather_k(table_hbm, idx_hbm, out_hbm, idx_v, rows_v, sem):
    wid = lax.axis_index("s") * NC + lax.axis_index("c")
    base = wid * b_per_w
    pltpu.sync_copy(idx_hbm.at[pl.ds(base, b_per_w)], idx_v)
    pltpu.async_copy(table_hbm.at[idx_v], rows_v, sem).wait()
    pltpu.sync_copy(rows_v, out_hbm.at[pl.ds(base, b_per_w)])

  return gather_k


def _qkv_body(xe_ref, mu_ref, var_ref, g_ref, b_ref, qp_ref, kp_ref, vp_ref,
              cos_ref, sin_ref, q_ref, k_ref, v_ref):
  x = xe_ref[...]
  mu = mu_ref[...]
  var = var_ref[...]
  h = (x - mu) / jnp.sqrt(var + 1e-5) * g_ref[...] + b_ref[...]
  c = cos_ref[...]
  s = sin_ref[...]

  def rope(t):
    parts = []
    for hh in range(NH):
      t1 = t[:, hh * DH:hh * DH + HALF]
      t2 = t[:, hh * DH + HALF:(hh + 1) * DH]
      parts.append(t1 * c - t2 * s)
      parts.append(t1 * s + t2 * c)
    return jnp.concatenate(parts, axis=1)

  q = jnp.dot(h, qp_ref[...], preferred_element_type=jnp.float32)
  k = jnp.dot(h, kp_ref[...], preferred_element_type=jnp.float32)
  q_ref[...] = rope(q)
  k_ref[...] = rope(k)
  v_ref[...] = jnp.dot(h, vp_ref[...], preferred_element_type=jnp.float32)


CKV = 1024


def _attn_body(q_ref, k_ref, v_ref, o_ref, *, bq):
  i = pl.program_id(1)
  q = q_ref[0]
  row = i * bq + lax.broadcasted_iota(jnp.int32, (bq, CKV), 0)
  m = jnp.full((bq, 1), NEG, jnp.float32)
  l = jnp.zeros((bq, 1), jnp.float32)
  acc = jnp.zeros((bq, DH), jnp.float32)
  for c in range(S // CKV):
    kc = k_ref[0, pl.ds(c * CKV, CKV), :]
    vc = v_ref[0, pl.ds(c * CKV, CKV), :]
    s = lax.dot_general(q, kc, (((1,), (1,)), ((), ())),
                        preferred_element_type=jnp.float32)
    s = s * (1.0 / math.sqrt(DH))
    col = c * CKV + lax.broadcasted_iota(jnp.int32, (bq, CKV), 1)
    s = jnp.where(col <= row, s, NEG)
    m_new = jnp.maximum(m, jnp.max(s, axis=1, keepdims=True))
    corr = jnp.exp(m - m_new)
    p = jnp.exp(s - m_new)
    acc = acc * corr + jnp.dot(p, vc, preferred_element_type=jnp.float32)
    l = l * corr + jnp.sum(p, axis=1, keepdims=True)
    m = m_new
  o_ref[0] = acc / l


def _proj_body(av_ref, op_ref, res_ref, attn_ref):
  attn_ref[...] = jnp.dot(av_ref[...], op_ref[...],
                          preferred_element_type=jnp.float32) + res_ref[...]


def _post_body(attn_ref, mu_ref, var_ref, g_ref, b_ref, gw_ref, gb_ref,
               normed_ref, logits_ref):
  normed = ((attn_ref[...] - mu_ref[...]) / jnp.sqrt(var_ref[...] + 1e-5)
            * g_ref[...] + b_ref[...])
  normed_ref[...] = normed
  logits_ref[...] = jnp.dot(normed, gw_ref[...],
                            preferred_element_type=jnp.float32) + gb_ref[...]


def _route_body(logits_ref, w_ref, aux_ref):
  lg = logits_ref[...]
  ie = lax.broadcasted_iota(jnp.int32, lg.shape, 1)
  m1 = jnp.max(lg, axis=1, keepdims=True)
  i1 = jnp.min(jnp.where(lg == m1, ie, E), axis=1, keepdims=True)
  lg2 = jnp.where(ie == i1, NEG, lg)
  m2 = jnp.max(lg2, axis=1, keepdims=True)
  i2 = jnp.min(jnp.where(lg2 == m2, ie, E), axis=1, keepdims=True)
  p1 = 1.0 / (1.0 + jnp.exp(m2 - m1))
  p2 = 1.0 - p1
  one1 = (ie == i1).astype(jnp.float32)
  one2 = (ie == i2).astype(jnp.float32)
  w_ref[...] = p1 * one1 + p2 * one2

  mfull = jnp.max(lg, axis=1, keepdims=True)
  ex = jnp.exp(lg - mfull)
  probs = ex / jnp.sum(ex, axis=1, keepdims=True)
  imp = jnp.sum(probs, axis=0, keepdims=True) * (1.0 / (B * S))
  cnt = jnp.sum(one1 + one2, axis=0, keepdims=True) * (1.0 / (B * S * K))
  aux_ref[...] = jnp.sum(imp * cnt, axis=1, keepdims=True) * E


def _moe_body(x_ref, w_ref, w1_ref, b1_ref, w2_ref, b2_ref, wd_ref, bd_ref,
              out_ref):
  e = pl.program_id(1)
  j = pl.program_id(2)
  x = x_ref[...]
  h1 = jnp.dot(x, w1_ref[0].astype(jnp.float32),
               preferred_element_type=jnp.float32) + b1_ref[0]
  h2 = jnp.dot(x, w2_ref[0].astype(jnp.float32),
               preferred_element_type=jnp.float32) + b2_ref[0]
  hid = h1 * jax.nn.sigmoid(h1) * h2
  part = jnp.dot(hid, wd_ref[0].astype(jnp.float32),
                 preferred_element_type=jnp.float32)
  ie = lax.broadcasted_iota(jnp.int32, w_ref.shape, 1)
  wcol = jnp.sum(jnp.where(ie == e, w_ref[...], 0.0), axis=1, keepdims=True)

  @pl.when(jnp.logical_and(e == 0, j == 0))
  def _():
    out_ref[...] = jnp.zeros_like(out_ref)

  contrib = wcol * part

  @pl.when(j == 0)
  def _():
    out_ref[...] += wcol * bd_ref[0]

  out_ref[...] += contrib


def kernel(x, emb, ln1_g, ln1_b, ln2_g, ln2_b, q_proj, k_proj, v_proj,
           out_proj, gate_W, gate_b, W1, b1, W2, b2, Wd, bd):
  f32 = jnp.float32
  xf = x.reshape(S).astype(jnp.int32)

  inv_freq = 1.0 / (10000.0 ** (jnp.arange(HALF, dtype=f32) / HALF))
  pos = jnp.arange(S, dtype=f32)
  ang = pos[:, None] * inv_freq[None, :]
  cos32, sin32 = jnp.cos(ang), jnp.sin(ang)
  perm = jnp.concatenate(
      [jnp.concatenate([jnp.arange(hh * DH, (hh + 1) * DH, 2),
                        jnp.arange(hh * DH + 1, (hh + 1) * DH, 2)])
       for hh in range(NH)])
  bf16 = jnp.bfloat16
  qp = q_proj[:, perm]
  kp = k_proj[:, perm]

  xe = _make_sc_gather(emb.shape[0], D, S, f32)(emb, xf)

  mu1 = jnp.mean(xe, axis=-1, keepdims=True)
  var1 = jnp.mean((xe - mu1) ** 2, axis=-1, keepdims=True)

  BT = 1024
  qkv = pl.pallas_call(
      _qkv_body,
      grid=(S // BT,),
      in_specs=[
          pl.BlockSpec((BT, D), lambda i: (i, 0)),
          pl.BlockSpec((BT, 1), lambda i: (i, 0)),
          pl.BlockSpec((BT, 1), lambda i: (i, 0)),
          pl.BlockSpec((1, D), lambda i: (0, 0)),
          pl.BlockSpec((1, D), lambda i: (0, 0)),
          pl.BlockSpec((D, D), lambda i: (0, 0)),
          pl.BlockSpec((D, D), lambda i: (0, 0)),
          pl.BlockSpec((D, D), lambda i: (0, 0)),
          pl.BlockSpec((BT, HALF), lambda i: (i, 0)),
          pl.BlockSpec((BT, HALF), lambda i: (i, 0)),
      ],
      out_specs=[pl.BlockSpec((BT, D), lambda i: (i, 0))] * 3,
      out_shape=[jax.ShapeDtypeStruct((S, D), f32)] * 3,
  )(xe, mu1, var1, ln1_g.reshape(1, D), ln1_b.reshape(1, D), qp, kp, v_proj,
    cos32, sin32)
  q, k, v = qkv
  qh = q.reshape(S, NH, DH).transpose(1, 0, 2)
  kh = k.reshape(S, NH, DH).transpose(1, 0, 2)
  vh = v.reshape(S, NH, DH).transpose(1, 0, 2)

  BQ = 256
  avh = pl.pallas_call(
      functools.partial(_attn_body, bq=BQ),
      grid=(NH, S // BQ),
      in_specs=[
          pl.BlockSpec((1, BQ, DH), lambda h, i: (h, i, 0)),
          pl.BlockSpec((1, S, DH), lambda h, i: (h, 0, 0)),
          pl.BlockSpec((1, S, DH), lambda h, i: (h, 0, 0)),
      ],
      out_specs=pl.BlockSpec((1, BQ, DH), lambda h, i: (h, i, 0)),
      out_shape=jax.ShapeDtypeStruct((NH, S, DH), f32),
  )(qh, kh, vh)
  av = avh.transpose(1, 0, 2).reshape(S, D)

  attn_out = pl.pallas_call(
      _proj_body,
      grid=(S // BT,),
      in_specs=[
          pl.BlockSpec((BT, D), lambda i: (i, 0)),
          pl.BlockSpec((D, D), lambda i: (0, 0)),
          pl.BlockSpec((BT, D), lambda i: (i, 0)),
      ],
      out_specs=pl.BlockSpec((BT, D), lambda i: (i, 0)),
      out_shape=jax.ShapeDtypeStruct((S, D), f32),
  )(av, out_proj, xe)
  mu2 = jnp.mean(attn_out, axis=-1, keepdims=True)
  var2 = jnp.mean((attn_out - mu2) ** 2, axis=-1, keepdims=True)
  normed, logits = pl.pallas_call(
      _post_body,
      grid=(S // BT,),
      in_specs=[
          pl.BlockSpec((BT, D), lambda i: (i, 0)),
          pl.BlockSpec((BT, 1), lambda i: (i, 0)),
          pl.BlockSpec((BT, 1), lambda i: (i, 0)),
          pl.BlockSpec((1, D), lambda i: (0, 0)),
          pl.BlockSpec((1, D), lambda i: (0, 0)),
          pl.BlockSpec((D, E), lambda i: (0, 0)),
          pl.BlockSpec((1, E), lambda i: (0, 0)),
      ],
      out_specs=[pl.BlockSpec((BT, D), lambda i: (i, 0)),
                 pl.BlockSpec((BT, E), lambda i: (i, 0))],
      out_shape=[jax.ShapeDtypeStruct((S, D), f32),
                 jax.ShapeDtypeStruct((S, E), f32)],
  )(attn_out, mu2, var2, ln2_g.reshape(1, D), ln2_b.reshape(1, D), gate_W,
    gate_b.reshape(1, E))

  wfull, aux = pl.pallas_call(
      _route_body,
      grid=(1,),
      in_specs=[pl.BlockSpec((S, E), lambda i: (0, 0))],
      out_specs=[pl.BlockSpec((S, E), lambda i: (0, 0)),
                 pl.BlockSpec((1, 1), lambda i: (0, 0))],
      out_shape=[jax.ShapeDtypeStruct((S, E), f32),
                 jax.ShapeDtypeStruct((1, 1), f32)],
  )(logits)

  BM, BH = 512, 512
  output = pl.pallas_call(
      _moe_body,
      grid=(S // BM, E, HID // BH),
      in_specs=[
          pl.BlockSpec((BM, D), lambda t, e, j: (t, 0)),
          pl.BlockSpec((BM, E), lambda t, e, j: (t, 0)),
          pl.BlockSpec((1, D, BH), lambda t, e, j: (e, 0, j)),
          pl.BlockSpec((1, 1, BH), lambda t, e, j: (e, 0, j)),
          pl.BlockSpec((1, D, BH), lambda t, e, j: (e, 0, j)),
          pl.BlockSpec((1, 1, BH), lambda t, e, j: (e, 0, j)),
          pl.BlockSpec((1, BH, D), lambda t, e, j: (e, j, 0)),
          pl.BlockSpec((1, 1, D), lambda t, e, j: (e, 0, 0)),
      ],
      out_specs=pl.BlockSpec((BM, D), lambda t, e, j: (t, 0)),
      out_shape=jax.ShapeDtypeStruct((S, D), f32),
      compiler_params=pltpu.CompilerParams(
          dimension_semantics=("arbitrary", "arbitrary", "arbitrary")),
  )(normed, wfull, W1.astype(bf16), b1.reshape(E, 1, HID),
    W2.astype(bf16), b2.reshape(E, 1, HID), Wd.astype(bf16),
    bd.reshape(E, 1, D))

  return output.reshape(B, S, D), aux.reshape(())


# speedup vs baseline: 1.1630x; 1.0624x over previous
import functools
import math

import jax
import jax.numpy as jnp
from jax import lax
from jax.experimental import pallas as pl
from jax.experimental.pallas import tpu as pltpu
from jax.experimental.pallas import tpu_sc as plsc

B, S, D = 1, 2048, 768
NH, DH = 12, 64
E, K, HID = 8, 2, 2048
MAX_LEN = 2048
HALF = DH // 2

NEG = -1e30


def _make_sc_gather(V, Dm, nrows, dtype):
  info = plsc.get_sparse_core_info()
  NC, NS = info.num_cores, info.num_subcores
  NW = NC * NS
  assert nrows % NW == 0
  b_per_w = nrows // NW
  mesh = plsc.VectorSubcoreMesh(core_axis_name="c", subcore_axis_name="s")

  @functools.partial(
      pl.kernel, mesh=mesh,
      out_type=jax.ShapeDtypeStruct((nrows, Dm), dtype),
      scratch_types=[
          pltpu.VMEM((b_per_w,), jnp.int32),
          pltpu.VMEM((b_per_w, Dm), dtype),
          pltpu.SemaphoreType.DMA,
      ],
  )
  def gather_k(table_hbm, idx_hbm, out_hbm, idx_v, rows_v, sem):
    wid = lax.axis_index("s") * NC + lax.axis_index("c")
    base = wid * b_per_w
    pltpu.sync_copy(idx_hbm.at[pl.ds(base, b_per_w)], idx_v)
    pltpu.async_copy(table_hbm.at[idx_v], rows_v, sem).wait()
    pltpu.sync_copy(rows_v, out_hbm.at[pl.ds(base, b_per_w)])

  return gather_k


def _qkv_body(xe_ref, mu_ref, var_ref, g_ref, b_ref, qp_ref, kp_ref, vp_ref,
              cos_ref, sin_ref, q_ref, k_ref, v_ref):
  x = xe_ref[...]
  mu = mu_ref[...]
  var = var_ref[...]
  h = (x - mu) / jnp.sqrt(var + 1e-5) * g_ref[...] + b_ref[...]
  c = cos_ref[...]
  s = sin_ref[...]

  def rope(t):
    parts = []
    for hh in range(NH):
      t1 = t[:, hh * DH:hh * DH + HALF]
      t2 = t[:, hh * DH + HALF:(hh + 1) * DH]
      parts.append(t1 * c - t2 * s)
      parts.append(t1 * s + t2 * c)
    return jnp.concatenate(parts, axis=1)

  q = jnp.dot(h, qp_ref[...], preferred_element_type=jnp.float32)
  k = jnp.dot(h, kp_ref[...], preferred_element_type=jnp.float32)
  q_ref[...] = rope(q)
  k_ref[...] = rope(k)
  v_ref[...] = jnp.dot(h, vp_ref[...], preferred_element_type=jnp.float32)


CKV = 1024


def _attn_body(q_ref, k_ref, v_ref, o_ref, *, bq):
  i = pl.program_id(1)
  q = q_ref[0]
  row = i * bq + lax.broadcasted_iota(jnp.int32, (bq, CKV), 0)
  m = jnp.full((bq, 1), NEG, jnp.float32)
  l = jnp.zeros((bq, 1), jnp.float32)
  acc = jnp.zeros((bq, DH), jnp.float32)
  for c in range(S // CKV):
    kc = k_ref[0, pl.ds(c * CKV, CKV), :]
    vc = v_ref[0, pl.ds(c * CKV, CKV), :]
    s = lax.dot_general(q, kc, (((1,), (1,)), ((), ())),
                        preferred_element_type=jnp.float32)
    s = s * (1.0 / math.sqrt(DH))
    col = c * CKV + lax.broadcasted_iota(jnp.int32, (bq, CKV), 1)
    s = jnp.where(col <= row, s, NEG)
    m_new = jnp.maximum(m, jnp.max(s, axis=1, keepdims=True))
    corr = jnp.exp(m - m_new)
    p = jnp.exp(s - m_new)
    acc = acc * corr + jnp.dot(p, vc, preferred_element_type=jnp.float32)
    l = l * corr + jnp.sum(p, axis=1, keepdims=True)
    m = m_new
  o_ref[0] = acc / l


def _proj_body(av_ref, op_ref, res_ref, attn_ref):
  attn_ref[...] = jnp.dot(av_ref[...], op_ref[...],
                          preferred_element_type=jnp.float32) + res_ref[...]


def _post_body(attn_ref, mu_ref, var_ref, g_ref, b_ref, gw_ref, gb_ref,
               normed_ref, logits_ref):
  normed = ((attn_ref[...] - mu_ref[...]) / jnp.sqrt(var_ref[...] + 1e-5)
            * g_ref[...] + b_ref[...])
  normed_ref[...] = normed
  logits_ref[...] = jnp.dot(normed, gw_ref[...],
                            preferred_element_type=jnp.float32) + gb_ref[...]


def _route_body(logits_ref, w_ref, aux_ref):
  lg = logits_ref[...]
  ie = lax.broadcasted_iota(jnp.int32, lg.shape, 1)
  m1 = jnp.max(lg, axis=1, keepdims=True)
  i1 = jnp.min(jnp.where(lg == m1, ie, E), axis=1, keepdims=True)
  lg2 = jnp.where(ie == i1, NEG, lg)
  m2 = jnp.max(lg2, axis=1, keepdims=True)
  i2 = jnp.min(jnp.where(lg2 == m2, ie, E), axis=1, keepdims=True)
  p1 = 1.0 / (1.0 + jnp.exp(m2 - m1))
  p2 = 1.0 - p1
  one1 = (ie == i1).astype(jnp.float32)
  one2 = (ie == i2).astype(jnp.float32)
  w_ref[...] = p1 * one1 + p2 * one2

  mfull = jnp.max(lg, axis=1, keepdims=True)
  ex = jnp.exp(lg - mfull)
  probs = ex / jnp.sum(ex, axis=1, keepdims=True)
  imp = jnp.sum(probs, axis=0, keepdims=True) * (1.0 / (B * S))
  cnt = jnp.sum(one1 + one2, axis=0, keepdims=True) * (1.0 / (B * S * K))
  aux_ref[...] = jnp.sum(imp * cnt, axis=1, keepdims=True) * E


def _moe_body(x_ref, w_ref, w1_ref, b1_ref, w2_ref, b2_ref, wd_ref, bd_ref,
              out_ref):
  e = pl.program_id(1)
  j = pl.program_id(2)
  x = x_ref[...]
  h1 = jnp.dot(x, w1_ref[0].astype(jnp.float32),
               preferred_element_type=jnp.float32) + b1_ref[0]
  h2 = jnp.dot(x, w2_ref[0].astype(jnp.float32),
               preferred_element_type=jnp.float32) + b2_ref[0]
  hid = h1 * jax.nn.sigmoid(h1) * h2
  part = jnp.dot(hid, wd_ref[0].astype(jnp.float32),
                 preferred_element_type=jnp.float32)
  ie = lax.broadcasted_iota(jnp.int32, w_ref.shape, 1)
  wcol = jnp.sum(jnp.where(ie == e, w_ref[...], 0.0), axis=1, keepdims=True)

  @pl.when(jnp.logical_and(e == 0, j == 0))
  def _():
    out_ref[...] = jnp.zeros_like(out_ref)

  contrib = wcol * part

  @pl.when(j == 0)
  def _():
    out_ref[...] += wcol * bd_ref[0]

  out_ref[...] += contrib


def kernel(x, emb, ln1_g, ln1_b, ln2_g, ln2_b, q_proj, k_proj, v_proj,
           out_proj, gate_W, gate_b, W1, b1, W2, b2, Wd, bd):
  f32 = jnp.float32
  xf = x.reshape(S).astype(jnp.int32)

  inv_freq = 1.0 / (10000.0 ** (jnp.arange(HALF, dtype=f32) / HALF))
  pos = jnp.arange(S, dtype=f32)
  ang = pos[:, None] * inv_freq[None, :]
  cos32, sin32 = jnp.cos(ang), jnp.sin(ang)
  perm = jnp.concatenate(
      [jnp.concatenate([jnp.arange(hh * DH, (hh + 1) * DH, 2),
                        jnp.arange(hh * DH + 1, (hh + 1) * DH, 2)])
       for hh in range(NH)])
  bf16 = jnp.bfloat16
  qp = q_proj[:, perm]
  kp = k_proj[:, perm]

  xe = _make_sc_gather(emb.shape[0], D, S, f32)(emb, xf)

  mu1 = jnp.mean(xe, axis=-1, keepdims=True)
  var1 = jnp.mean((xe - mu1) ** 2, axis=-1, keepdims=True)

  BT = 1024
  qkv = pl.pallas_call(
      _qkv_body,
      grid=(S // BT,),
      in_specs=[
          pl.BlockSpec((BT, D), lambda i: (i, 0)),
          pl.BlockSpec((BT, 1), lambda i: (i, 0)),
          pl.BlockSpec((BT, 1), lambda i: (i, 0)),
          pl.BlockSpec((1, D), lambda i: (0, 0)),
          pl.BlockSpec((1, D), lambda i: (0, 0)),
          pl.BlockSpec((D, D), lambda i: (0, 0)),
          pl.BlockSpec((D, D), lambda i: (0, 0)),
          pl.BlockSpec((D, D), lambda i: (0, 0)),
          pl.BlockSpec((BT, HALF), lambda i: (i, 0)),
          pl.BlockSpec((BT, HALF), lambda i: (i, 0)),
      ],
      out_specs=[pl.BlockSpec((BT, D), lambda i: (i, 0))] * 3,
      out_shape=[jax.ShapeDtypeStruct((S, D), f32)] * 3,
  )(xe, mu1, var1, ln1_g.reshape(1, D), ln1_b.reshape(1, D), qp, kp, v_proj,
    cos32, sin32)
  q, k, v = qkv
  qh = q.reshape(S, NH, DH).transpose(1, 0, 2)
  kh = k.reshape(S, NH, DH).transpose(1, 0, 2)
  vh = v.reshape(S, NH, DH).transpose(1, 0, 2)

  BQ = 256
  avh = pl.pallas_call(
      functools.partial(_attn_body, bq=BQ),
      grid=(NH, S // BQ),
      in_specs=[
          pl.BlockSpec((1, BQ, DH), lambda h, i: (h, i, 0)),
          pl.BlockSpec((1, S, DH), lambda h, i: (h, 0, 0)),
          pl.BlockSpec((1, S, DH), lambda h, i: (h, 0, 0)),
      ],
      out_specs=pl.BlockSpec((1, BQ, DH), lambda h, i: (h, i, 0)),
      out_shape=jax.ShapeDtypeStruct((NH, S, DH), f32),
  )(qh, kh, vh)
  av = avh.transpose(1, 0, 2).reshape(S, D)

  attn_out = pl.pallas_call(
      _proj_body,
      grid=(S // BT,),
      in_specs=[
          pl.BlockSpec((BT, D), lambda i: (i, 0)),
          pl.BlockSpec((D, D), lambda i: (0, 0)),
          pl.BlockSpec((BT, D), lambda i: (i, 0)),
      ],
      out_specs=pl.BlockSpec((BT, D), lambda i: (i, 0)),
      out_shape=jax.ShapeDtypeStruct((S, D), f32),
  )(av, out_proj, xe)
  mu2 = jnp.mean(attn_out, axis=-1, keepdims=True)
  var2 = jnp.mean((attn_out - mu2) ** 2, axis=-1, keepdims=True)
  normed, logits = pl.pallas_call(
      _post_body,
      grid=(S // BT,),
      in_specs=[
          pl.BlockSpec((BT, D), lambda i: (i, 0)),
          pl.BlockSpec((BT, 1), lambda i: (i, 0)),
          pl.BlockSpec((BT, 1), lambda i: (i, 0)),
          pl.BlockSpec((1, D), lambda i: (0, 0)),
          pl.BlockSpec((1, D), lambda i: (0, 0)),
          pl.BlockSpec((D, E), lambda i: (0, 0)),
          pl.BlockSpec((1, E), lambda i: (0, 0)),
      ],
      out_specs=[pl.BlockSpec((BT, D), lambda i: (i, 0)),
                 pl.BlockSpec((BT, E), lambda i: (i, 0))],
      out_shape=[jax.ShapeDtypeStruct((S, D), f32),
                 jax.ShapeDtypeStruct((S, E), f32)],
  )(attn_out, mu2, var2, ln2_g.reshape(1, D), ln2_b.reshape(1, D), gate_W,
    gate_b.reshape(1, E))

  wfull, aux = pl.pallas_call(
      _route_body,
      grid=(1,),
      in_specs=[pl.BlockSpec((S, E), lambda i: (0, 0))],
      out_specs=[pl.BlockSpec((S, E), lambda i: (0, 0)),
                 pl.BlockSpec((1, 1), lambda i: (0, 0))],
      out_shape=[jax.ShapeDtypeStruct((S, E), f32),
                 jax.ShapeDtypeStruct((1, 1), f32)],
  )(logits)

  BM, BH = 2048, 512
  output = pl.pallas_call(
      _moe_body,
      grid=(S // BM, E, HID // BH),
      in_specs=[
          pl.BlockSpec((BM, D), lambda t, e, j: (t, 0)),
          pl.BlockSpec((BM, E), lambda t, e, j: (t, 0)),
          pl.BlockSpec((1, D, BH), lambda t, e, j: (e, 0, j)),
          pl.BlockSpec((1, 1, BH), lambda t, e, j: (e, 0, j)),
          pl.BlockSpec((1, D, BH), lambda t, e, j: (e, 0, j)),
          pl.BlockSpec((1, 1, BH), lambda t, e, j: (e, 0, j)),
          pl.BlockSpec((1, BH, D), lambda t, e, j: (e, j, 0)),
          pl.BlockSpec((1, 1, D), lambda t, e, j: (e, 0, 0)),
      ],
      out_specs=pl.BlockSpec((BM, D), lambda t, e, j: (t, 0)),
      out_shape=jax.ShapeDtypeStruct((S, D), f32),
      compiler_params=pltpu.CompilerParams(
          dimension_semantics=("arbitrary", "arbitrary", "arbitrary")),
  )(normed, wfull, W1.astype(bf16), b1.reshape(E, 1, HID),
    W2.astype(bf16), b2.reshape(E, 1, HID), Wd.astype(bf16),
    bd.reshape(E, 1, D))

  return output.reshape(B, S, D), aux.reshape(())
